# baseline (speedup 1.0000x reference)
import functools

import jax
import jax.numpy as jnp
from jax import lax
from jax.experimental import pallas as pl
from jax.experimental.pallas import tpu as pltpu
from jax.experimental.pallas import tpu_sc as plsc

_L = 16


def _skey(f):
    i = lax.bitcast_convert_type(f, jnp.int32)
    return i ^ (lax.shift_right_arithmetic(i, 31) & jnp.int32(0x7FFFFFFF))


def _sc_select_kernel(batch, n, k, rows_per_worker, interpret=False):
    nvec = n // _L
    nbins = 1024
    nhist_vec = nbins // _L
    shift = 22
    mesh = plsc.VectorSubcoreMesh(core_axis_name="c", subcore_axis_name="s")
    num_workers = batch // rows_per_worker

    @functools.partial(
        pl.kernel,
        out_type=jax.ShapeDtypeStruct((num_workers, _L), jnp.int32),
        mesh=mesh,
        scratch_types=[
            pltpu.VMEM((n,), jnp.int32),
            pltpu.VMEM((n,), jnp.float32),
            pltpu.VMEM((n,), jnp.int32),
            pltpu.VMEM((nbins * _L,), jnp.int32),
            pltpu.VMEM((_L,), jnp.int32),
        ],
        compiler_params=pltpu.CompilerParams(needs_layout_passes=False),
        interpret=interpret,
    )
    def sc_select(xi_hbm, duty_hbm, out_hbm, buf, fac, cand, hist, tbuf):
        wid = lax.axis_index("s") * 2 + lax.axis_index("c")
        lane = lax.iota(jnp.int32, _L)
        ones = jnp.ones((_L,), jnp.int32)
        zeros = jnp.zeros((_L,), jnp.int32)

        pltpu.sync_copy(duty_hbm, fac)

        def exp_body(i, _):
            sl = pl.ds(i * _L, _L)
            fac[sl] = jnp.exp(-fac[sl])
            return 0

        lax.fori_loop(0, nvec, exp_body, 0)

        def row_body(r, tvec):
            row = wid * rows_per_worker + r
            pltpu.sync_copy(xi_hbm.at[row], buf)

            def hclr(j, _):
                hist[pl.ds(j * _L, _L)] = zeros
                return 0

            lax.fori_loop(0, nbins, hclr, 0)

            def p1(i, _):
                sl = pl.ds(i * _L, _L)
                xf = plsc.bitcast(buf[sl], jnp.float32)
                key = _skey(xf * fac[sl])
                buf[sl] = key
                bin_ = lax.shift_right_arithmetic(key, shift) + jnp.int32(512)
                idx = bin_ * _L + lane
                plsc.addupdate_scatter(hist, [idx], ones)
                return 0

            lax.fori_loop(0, nvec, p1, 0)

            def bf(t, carry):
                s_above, b_run, a_run = carry
                j = nhist_vec - 1 - t
                base = (jnp.int32(j * _L) + lane) * _L
                tot = zeros
                for l in range(_L):
                    tot = tot + plsc.load_gather(hist, [base + jnp.int32(l)])
                s = lax.rev(jnp.cumsum(lax.rev(tot, (0,))), (0,)) + s_above
                m = s >= jnp.int32(k)
                candb = jnp.max(jnp.where(m, jnp.int32(j * _L) + lane, jnp.int32(-1)))
                canda = jnp.min(jnp.where(m, s - tot, jnp.int32(2**30)))
                a_new = jnp.where(candb > b_run, canda, a_run)
                b_new = jnp.maximum(b_run, candb)
                return s_above + jnp.sum(tot), b_new, a_new

            _, b_star, a_above = lax.fori_loop(
                0, nhist_vec, bf, (jnp.int32(0), jnp.int32(-1), jnp.int32(0))
            )
            rank = jnp.int32(k) - a_above

            def p2(i, off):
                sl = pl.ds(i * _L, _L)
                key = buf[sl]
                bin_ = lax.shift_right_arithmetic(key, shift) + jnp.int32(512)
                m = bin_ == b_star
                plsc.store_compressed(cand.at[pl.ds(off, _L)], key, mask=m)
                return off + jnp.sum(m.astype(jnp.int32))

            ncand = lax.fori_loop(0, nvec, p2, jnp.int32(0))
            ncvec = (ncand + jnp.int32(_L - 1)) // jnp.int32(_L)

            prefix0 = lax.shift_left(b_star - jnp.int32(512), jnp.int32(shift))

            def bs(b, prefix):
                trial = prefix + lax.shift_left(
                    jnp.int32(1), jnp.int32(shift - 1) - b
                )

                def cnt_body(i, c):
                    sl = pl.ds(i * _L, _L)
                    inb = (i * _L + lane) < ncand
                    ge = (cand[sl] >= trial) & inb
                    return c + jnp.sum(ge.astype(jnp.int32))

                cnt = lax.fori_loop(0, ncvec, cnt_body, jnp.int32(0))
                return jnp.where(cnt >= rank, trial, prefix)

            tkey = lax.fori_loop(0, shift, bs, prefix0)
            return jnp.where(lane == r, tkey, tvec)

        tvec = lax.fori_loop(0, rows_per_worker, row_body, zeros)
        tbuf[...] = tvec
        pltpu.sync_copy(tbuf, out_hbm.at[wid])

    return sc_select


def _mask_block(x_ref, duty_ref, t_ref, o_ref):
    x = x_ref[...]
    boosted = x * jnp.exp(-duty_ref[...])
    skey = _skey(boosted)
    o_ref[...] = jnp.where(skey < t_ref[...], jnp.zeros_like(x), x)


def kernel(x, duty_cycles):
    batch, n = x.shape
    k = int(round(n * 0.25))
    rows_per_worker = batch // 32

    xi = lax.bitcast_convert_type(x, jnp.int32)
    sc_out = _sc_select_kernel(batch, n, k, rows_per_worker)(xi, duty_cycles)
    tkeys = sc_out[:, :rows_per_worker].reshape(batch, 1)

    rows_per_block = 8
    duty2 = duty_cycles.reshape(1, n)
    return pl.pallas_call(
        _mask_block,
        grid=(batch // rows_per_block,),
        in_specs=[
            pl.BlockSpec((rows_per_block, n), lambda i: (i, 0)),
            pl.BlockSpec((1, n), lambda i: (0, 0)),
            pl.BlockSpec((rows_per_block, 1), lambda i: (i, 0)),
        ],
        out_specs=pl.BlockSpec((rows_per_block, n), lambda i: (i, 0)),
        out_shape=jax.ShapeDtypeStruct((batch, n), x.dtype),
    )(x, duty2, tkeys)


# speedup vs baseline: 1.2824x; 1.2824x over previous
import functools

import jax
import jax.numpy as jnp
from jax import lax
from jax.experimental import pallas as pl
from jax.experimental.pallas import tpu as pltpu
from jax.experimental.pallas import tpu_sc as plsc

_L = 16


def _skey(f):
    i = lax.bitcast_convert_type(f, jnp.int32)
    return i ^ (lax.shift_right_arithmetic(i, 31) & jnp.int32(0x7FFFFFFF))


def _sc_select_kernel(batch, n, k, rows_per_worker, interpret=False):
    nvec = n // _L
    nbins = 1024
    nhist_vec = nbins // _L
    shift = 22
    unroll = 4
    mesh = plsc.VectorSubcoreMesh(core_axis_name="c", subcore_axis_name="s")
    num_workers = batch // rows_per_worker

    @functools.partial(
        pl.kernel,
        out_type=jax.ShapeDtypeStruct((num_workers, _L), jnp.int32),
        mesh=mesh,
        scratch_types=[
            pltpu.VMEM((n,), jnp.int32),
            pltpu.VMEM((n,), jnp.int32),
            pltpu.VMEM((n,), jnp.float32),
            pltpu.VMEM((nbins * _L,), jnp.int32),
            pltpu.VMEM((_L,), jnp.int32),
            pltpu.SemaphoreType.DMA,
            pltpu.SemaphoreType.DMA,
        ],
        compiler_params=pltpu.CompilerParams(needs_layout_passes=False),
        interpret=interpret,
    )
    def sc_select(xi_hbm, duty_hbm, out_hbm, buf_a, buf_b, fac, hist, tbuf,
                  sem0, sem1):
        wid = lax.axis_index("s") * 2 + lax.axis_index("c")
        lane = lax.iota(jnp.int32, _L)
        ones = jnp.ones((_L,), jnp.int32)
        zeros = jnp.zeros((_L,), jnp.int32)
        sems = (sem0, sem1)
        bufs = (buf_a, buf_b)

        row0 = wid * rows_per_worker
        cp0 = pltpu.async_copy(xi_hbm.at[row0], bufs[0], sems[0])

        pltpu.sync_copy(duty_hbm, fac)

        def exp_body(i, _):
            for u in range(unroll):
                sl = pl.ds((i * unroll + u) * _L, _L)
                fac[sl] = jnp.exp(-fac[sl])
            return 0

        lax.fori_loop(0, nvec // unroll, exp_body, 0)

        cp0.wait()
        tvec = zeros
        for r in range(rows_per_worker):
            buf = bufs[r % 2]
            if r + 1 < rows_per_worker:
                nxt = pltpu.async_copy(
                    xi_hbm.at[row0 + r + 1], bufs[(r + 1) % 2],
                    sems[(r + 1) % 2],
                )

            def hclr(j, _):
                for u in range(unroll):
                    hist[pl.ds((j * unroll + u) * _L, _L)] = zeros
                return 0

            lax.fori_loop(0, nbins // unroll, hclr, 0)

            def p1(i, _):
                for u in range(unroll):
                    sl = pl.ds((i * unroll + u) * _L, _L)
                    xf = plsc.bitcast(buf[sl], jnp.float32)
                    key = _skey(xf * fac[sl])
                    buf[sl] = key
                    bin_ = lax.shift_right_arithmetic(key, shift) + jnp.int32(512)
                    idx = bin_ * _L + lane
                    plsc.addupdate_scatter(hist, [idx], ones)
                return 0

            lax.fori_loop(0, nvec // unroll, p1, 0)

            def bf(t, carry):
                s_above, b_run, a_run = carry
                j = nhist_vec - 1 - t
                base = (jnp.int32(j * _L) + lane) * _L
                tot = zeros
                for l in range(_L):
                    tot = tot + plsc.load_gather(hist, [base + jnp.int32(l)])
                s = lax.rev(jnp.cumsum(lax.rev(tot, (0,))), (0,)) + s_above
                m = s >= jnp.int32(k)
                candb = jnp.max(
                    jnp.where(m, jnp.int32(j * _L) + lane, jnp.int32(-1)))
                canda = jnp.min(jnp.where(m, s - tot, jnp.int32(2**30)))
                a_new = jnp.where(candb > b_run, canda, a_run)
                b_new = jnp.maximum(b_run, candb)
                return s_above + jnp.sum(tot), b_new, a_new

            _, b_star, a_above = lax.fori_loop(
                0, nhist_vec, bf, (jnp.int32(0), jnp.int32(-1), jnp.int32(0))
            )
            rank = jnp.int32(k) - a_above

            def p2(i, off):
                for u in range(unroll):
                    sl = pl.ds((i * unroll + u) * _L, _L)
                    key = buf[sl]
                    bin_ = lax.shift_right_arithmetic(key, shift) + jnp.int32(512)
                    m = bin_ == b_star
                    plsc.store_compressed(buf.at[pl.ds(off, _L)], key, mask=m)
                    off = off + plsc.all_reduce_population_count(m)[0]
                return off

            ncand = lax.fori_loop(0, nvec // unroll, p2, jnp.int32(0))
            ncvec = (ncand + jnp.int32(_L - 1)) // jnp.int32(_L)

            prefix0 = jnp.full((_L,), 1, jnp.int32) * lax.shift_left(
                b_star - jnp.int32(512), jnp.int32(shift))
            rank_v = jnp.full((_L,), 1, jnp.int32) * rank
            ncand_v = jnp.full((_L,), 1, jnp.int32) * ncand

            def bs(b, prefix):
                trial = prefix + lax.shift_left(
                    jnp.int32(1), jnp.int32(shift - 1) - b)

                def cnt_body(i, c):
                    for u in range(unroll):
                        iu = i * unroll + u
                        sl = pl.ds(iu * _L, _L)
                        inb = (iu * _L + lane) < ncand_v
                        ge = (buf[sl] >= trial) & inb
                        c = c + plsc.all_reduce_population_count(ge)
                    return c

                cnt = lax.fori_loop(0, (ncvec + unroll - 1) // unroll,
                                    cnt_body, zeros)
                return jnp.where(cnt >= rank_v, trial, prefix)

            tkey_v = lax.fori_loop(0, shift, bs, prefix0)
            tvec = jnp.where(lane == r, tkey_v[0], tvec)
            if r + 1 < rows_per_worker:
                nxt.wait()

        tbuf[...] = tvec
        pltpu.sync_copy(tbuf, out_hbm.at[wid])

    return sc_select


def _mask_block(x_ref, duty_ref, t_ref, o_ref):
    x = x_ref[...]
    boosted = x * jnp.exp(-duty_ref[...])
    skey = _skey(boosted)
    o_ref[...] = jnp.where(skey < t_ref[...], jnp.zeros_like(x), x)


def kernel(x, duty_cycles):
    batch, n = x.shape
    k = int(round(n * 0.25))
    rows_per_worker = batch // 32

    xi = lax.bitcast_convert_type(x, jnp.int32)
    sc_out = _sc_select_kernel(batch, n, k, rows_per_worker)(xi, duty_cycles)
    tkeys = sc_out[:, :rows_per_worker].reshape(batch, 1)

    rows_per_block = 8
    duty2 = duty_cycles.reshape(1, n)
    return pl.pallas_call(
        _mask_block,
        grid=(batch // rows_per_block,),
        in_specs=[
            pl.BlockSpec((rows_per_block, n), lambda i: (i, 0)),
            pl.BlockSpec((1, n), lambda i: (0, 0)),
            pl.BlockSpec((rows_per_block, 1), lambda i: (i, 0)),
        ],
        out_specs=pl.BlockSpec((rows_per_block, n), lambda i: (i, 0)),
        out_shape=jax.ShapeDtypeStruct((batch, n), x.dtype),
    )(x, duty2, tkeys)


# speedup vs baseline: 2.2988x; 1.7925x over previous
import functools

import jax
import jax.numpy as jnp
from jax import lax
from jax.experimental import pallas as pl
from jax.experimental.pallas import tpu as pltpu
from jax.experimental.pallas import tpu_sc as plsc

_L = 16


def _skey(f):
    i = lax.bitcast_convert_type(f, jnp.int32)
    return i ^ (lax.shift_right_arithmetic(i, 31) & jnp.int32(0x7FFFFFFF))


def _sc_select_kernel(batch, n, k, rows_per_worker, interpret=False):
    nvec = n // _L
    nbins = 1024
    nhist_vec = nbins // _L
    shift = 22
    mesh = plsc.VectorSubcoreMesh(core_axis_name="c", subcore_axis_name="s")
    num_workers = batch // rows_per_worker

    @functools.partial(
        pl.kernel,
        out_type=jax.ShapeDtypeStruct((num_workers, _L), jnp.int32),
        mesh=mesh,
        scratch_types=[
            pltpu.VMEM((n,), jnp.int32),
            pltpu.VMEM((n,), jnp.int32),
            pltpu.VMEM((n,), jnp.float32),
            pltpu.VMEM((nbins * _L,), jnp.int32),
            pltpu.VMEM((nvec + _L,), jnp.int32),
            pltpu.VMEM((_L,), jnp.int32),
        ],
        compiler_params=pltpu.CompilerParams(needs_layout_passes=False),
        interpret=interpret,
    )
    def sc_select(xi_hbm, duty_hbm, out_hbm, buf, cand, fac, hist, pcnt, tbuf):
        wid = lax.axis_index("s") * 2 + lax.axis_index("c")
        lane = lax.iota(jnp.int32, _L)
        ones = jnp.ones((_L,), jnp.int32)
        zeros = jnp.zeros((_L,), jnp.int32)
        lane0 = lane == 0

        pltpu.sync_copy(duty_hbm, fac)

        @plsc.parallel_loop(0, nvec, unroll=8)
        def _(i):
            sl = pl.ds(i * _L, _L)
            fac[sl] = jnp.exp(-fac[sl])

        tvec = zeros
        for r in range(rows_per_worker):
            row = wid * rows_per_worker + r
            pltpu.sync_copy(xi_hbm.at[row], buf)

            @plsc.parallel_loop(0, nbins, unroll=8)
            def _(j):
                hist[pl.ds(j * _L, _L)] = zeros

            @plsc.parallel_loop(0, nvec, unroll=8)
            def _(i):
                sl = pl.ds(i * _L, _L)
                xf = plsc.bitcast(buf[sl], jnp.float32)
                key = _skey(xf * fac[sl])
                buf[sl] = key
                bin_ = lax.shift_right_arithmetic(key, shift) + jnp.int32(512)
                plsc.addupdate_scatter(hist, [bin_ * _L + lane], ones)

            def bf(t, carry):
                s_above, b_run, a_run = carry
                j = nhist_vec - 1 - t
                base = (jnp.int32(j * _L) + lane) * _L
                tot = zeros
                for l in range(_L):
                    tot = tot + plsc.load_gather(hist, [base + jnp.int32(l)])
                s = lax.rev(jnp.cumsum(lax.rev(tot, (0,))), (0,)) + s_above
                m = s >= jnp.int32(k)
                candb = jnp.max(
                    jnp.where(m, jnp.int32(j * _L) + lane, jnp.int32(-1)))
                canda = jnp.min(jnp.where(m, s - tot, jnp.int32(2**30)))
                a_new = jnp.where(candb > b_run, canda, a_run)
                b_new = jnp.maximum(b_run, candb)
                return s_above + jnp.sum(tot), b_new, a_new

            _, b_star, a_above = lax.fori_loop(
                0, nhist_vec, bf, (jnp.int32(0), jnp.int32(-1), jnp.int32(0))
            )
            rank = jnp.int32(k) - a_above
            bin_lo = lax.shift_left(b_star - jnp.int32(512), jnp.int32(shift))

            @plsc.parallel_loop(0, nvec, unroll=8)
            def _(i):
                sl = pl.ds(i * _L, _L)
                bin_ = lax.shift_right_arithmetic(buf[sl], shift) + jnp.int32(512)
                pc = plsc.all_reduce_population_count(bin_ == b_star)
                plsc.store_compressed(pcnt.at[pl.ds(i, _L)], pc, mask=lane0)

            def scan_body(j, carry):
                sl = pl.ds(j * _L, _L)
                c = pcnt[sl]
                cs = jnp.cumsum(c)
                pcnt[sl] = cs - c + carry
                return carry + jnp.max(
                    jnp.where(lane == jnp.int32(_L - 1), cs, jnp.int32(0)))

            ncand = lax.fori_loop(0, nvec // _L, scan_body, jnp.int32(0))
            ncvec = (ncand + jnp.int32(_L - 1)) // jnp.int32(_L)

            @plsc.parallel_loop(0, nvec, unroll=4)
            def _(i):
                sl = pl.ds(i * _L, _L)
                key = buf[sl]
                m = (lax.shift_right_arithmetic(key, shift)
                     + jnp.int32(512)) == b_star
                off = pcnt[pl.ds(i, _L)][0]
                plsc.store_compressed(cand.at[pl.ds(off, _L)], key, mask=m)

            rank_v = jnp.broadcast_to(rank, (_L,))
            ncand_v = jnp.broadcast_to(ncand, (_L,))

            def bs(b, prefix):
                trial = prefix + lax.shift_left(
                    jnp.int32(1), jnp.int32(shift - 1) - b)
                trial_v = jnp.broadcast_to(trial, (_L,))

                @plsc.parallel_loop(0, ncvec, unroll=4, carry=zeros)
                def cnt(i, c):
                    inb = (i * _L + lane) < ncand_v
                    ge = (cand[pl.ds(i * _L, _L)] >= trial_v) & inb
                    return c + plsc.all_reduce_population_count(ge)

                return jnp.where(jnp.max(cnt) >= rank, trial, prefix)

            tkey = lax.fori_loop(0, shift, bs, bin_lo)
            tvec = jnp.where(lane == r, tkey, tvec)

        tbuf[...] = tvec
        pltpu.sync_copy(tbuf, out_hbm.at[wid])

    return sc_select


def _mask_block(x_ref, duty_ref, t_ref, o_ref):
    x = x_ref[...]
    boosted = x * jnp.exp(-duty_ref[...])
    skey = _skey(boosted)
    o_ref[...] = jnp.where(skey < t_ref[...], jnp.zeros_like(x), x)


def kernel(x, duty_cycles):
    batch, n = x.shape
    k = int(round(n * 0.25))
    rows_per_worker = batch // 32

    xi = lax.bitcast_convert_type(x, jnp.int32)
    sc_out = _sc_select_kernel(batch, n, k, rows_per_worker)(xi, duty_cycles)
    tkeys = sc_out[:, :rows_per_worker].reshape(batch, 1)

    rows_per_block = 8
    duty2 = duty_cycles.reshape(1, n)
    return pl.pallas_call(
        _mask_block,
        grid=(batch // rows_per_block,),
        in_specs=[
            pl.BlockSpec((rows_per_block, n), lambda i: (i, 0)),
            pl.BlockSpec((1, n), lambda i: (0, 0)),
            pl.BlockSpec((rows_per_block, 1), lambda i: (i, 0)),
        ],
        out_specs=pl.BlockSpec((rows_per_block, n), lambda i: (i, 0)),
        out_shape=jax.ShapeDtypeStruct((batch, n), x.dtype),
    )(x, duty2, tkeys)


# speedup vs baseline: 2.4284x; 1.0564x over previous
import functools

import jax
import jax.numpy as jnp
from jax import lax
from jax.experimental import pallas as pl
from jax.experimental.pallas import tpu as pltpu
from jax.experimental.pallas import tpu_sc as plsc

_L = 16


def _skey(f):
    i = lax.bitcast_convert_type(f, jnp.int32)
    return i ^ (lax.shift_right_arithmetic(i, 31) & jnp.int32(0x7FFFFFFF))


def _sc_select_kernel(batch, n, k, rows_per_worker, interpret=False):
    nvec = n // _L
    nbins = 1024
    nhist_vec = nbins // _L
    shift = 22
    mesh = plsc.VectorSubcoreMesh(core_axis_name="c", subcore_axis_name="s")
    num_workers = batch // rows_per_worker

    @functools.partial(
        pl.kernel,
        out_type=jax.ShapeDtypeStruct((num_workers, _L), jnp.int32),
        mesh=mesh,
        scratch_types=[
            pltpu.VMEM((n,), jnp.float32),
            pltpu.VMEM((n,), jnp.int32),
            pltpu.VMEM((n,), jnp.float32),
            pltpu.VMEM((nbins * _L,), jnp.int32),
            pltpu.VMEM((nvec + _L,), jnp.int32),
            pltpu.VMEM((_L,), jnp.int32),
        ],
        compiler_params=pltpu.CompilerParams(needs_layout_passes=False),
        interpret=interpret,
    )
    def sc_select(x_hbm, duty_hbm, out_hbm, buf, cand, fac, hist, pcnt, tbuf):
        wid = lax.axis_index("s") * 2 + lax.axis_index("c")
        lane = lax.iota(jnp.int32, _L)
        ones = jnp.ones((_L,), jnp.int32)
        zeros = jnp.zeros((_L,), jnp.int32)
        lane0 = lane == 0

        pltpu.sync_copy(duty_hbm, fac)

        @plsc.parallel_loop(0, nvec, unroll=8)
        def _(i):
            sl = pl.ds(i * _L, _L)
            fac[sl] = jnp.exp(-fac[sl])

        tvec = zeros
        for r in range(rows_per_worker):
            row = wid * rows_per_worker + r
            pltpu.sync_copy(x_hbm.at[row], buf)

            @plsc.parallel_loop(0, nbins, unroll=8)
            def _(j):
                hist[pl.ds(j * _L, _L)] = zeros

            @plsc.parallel_loop(0, nvec, unroll=8)
            def _(i):
                sl = pl.ds(i * _L, _L)
                key = _skey(buf[sl] * fac[sl])
                buf[sl] = plsc.bitcast(key, jnp.float32)
                bin_ = lax.shift_right_arithmetic(key, shift) + jnp.int32(512)
                plsc.addupdate_scatter(hist, [bin_ * _L + lane], ones)

            def bf(t, carry):
                s_above, b_run, a_run = carry
                j = nhist_vec - 1 - t
                base = (jnp.int32(j * _L) + lane) * _L
                tot = zeros
                for l in range(_L):
                    tot = tot + plsc.load_gather(hist, [base + jnp.int32(l)])
                s = lax.rev(jnp.cumsum(lax.rev(tot, (0,))), (0,)) + s_above
                m = s >= jnp.int32(k)
                candb = jnp.max(
                    jnp.where(m, jnp.int32(j * _L) + lane, jnp.int32(-1)))
                canda = jnp.min(jnp.where(m, s - tot, jnp.int32(2**30)))
                a_new = jnp.where(candb > b_run, canda, a_run)
                b_new = jnp.maximum(b_run, candb)
                return s_above + jnp.sum(tot), b_new, a_new

            _, b_star, a_above = lax.fori_loop(
                0, nhist_vec, bf, (jnp.int32(0), jnp.int32(-1), jnp.int32(0))
            )
            rank = jnp.int32(k) - a_above
            bin_lo = lax.shift_left(b_star - jnp.int32(512), jnp.int32(shift))

            @plsc.parallel_loop(0, nvec, unroll=8)
            def _(i):
                sl = pl.ds(i * _L, _L)
                key = plsc.bitcast(buf[sl], jnp.int32)
                bin_ = lax.shift_right_arithmetic(key, shift) + jnp.int32(512)
                pc = plsc.all_reduce_population_count(bin_ == b_star)
                plsc.store_compressed(pcnt.at[pl.ds(i, _L)], pc, mask=lane0)

            def scan_body(j, carry):
                sl = pl.ds(j * _L, _L)
                c = pcnt[sl]
                cs = jnp.cumsum(c)
                pcnt[sl] = cs - c + carry
                return carry + jnp.max(
                    jnp.where(lane == jnp.int32(_L - 1), cs, jnp.int32(0)))

            ncand = lax.fori_loop(0, nvec // _L, scan_body, jnp.int32(0))
            ncvec = (ncand + jnp.int32(_L - 1)) // jnp.int32(_L)

            @plsc.parallel_loop(0, nvec, unroll=4)
            def _(i):
                sl = pl.ds(i * _L, _L)
                key = plsc.bitcast(buf[sl], jnp.int32)
                m = (lax.shift_right_arithmetic(key, shift)
                     + jnp.int32(512)) == b_star
                off = pcnt[pl.ds(i, _L)][0]
                plsc.store_compressed(cand.at[pl.ds(off, _L)], key, mask=m)

            rank_v = jnp.broadcast_to(rank, (_L,))
            ncand_v = jnp.broadcast_to(ncand, (_L,))

            def bs(b, prefix):
                trial = prefix + lax.shift_left(
                    jnp.int32(1), jnp.int32(shift - 1) - b)
                trial_v = jnp.broadcast_to(trial, (_L,))

                @plsc.parallel_loop(0, ncvec, unroll=4, carry=zeros)
                def cnt(i, c):
                    inb = (i * _L + lane) < ncand_v
                    ge = (cand[pl.ds(i * _L, _L)] >= trial_v) & inb
                    return c + plsc.all_reduce_population_count(ge)

                return jnp.where(jnp.max(cnt) >= rank, trial, prefix)

            tkey = lax.fori_loop(0, shift, bs, bin_lo)
            tvec = jnp.where(lane == r, tkey, tvec)

        tbuf[...] = tvec
        pltpu.sync_copy(tbuf, out_hbm.at[wid])

    return sc_select


def _mask_block(x_ref, duty_ref, t_ref, o_ref):
    x = x_ref[...]
    boosted = x * jnp.exp(-duty_ref[...])
    skey = _skey(boosted)
    o_ref[...] = jnp.where(skey < t_ref[...], jnp.zeros_like(x), x)


def kernel(x, duty_cycles):
    batch, n = x.shape
    k = int(round(n * 0.25))
    rows_per_worker = batch // 32

    sc_out = _sc_select_kernel(batch, n, k, rows_per_worker)(x, duty_cycles)
    tkeys = sc_out[:, :rows_per_worker].reshape(batch, 1)

    rows_per_block = 8
    duty2 = duty_cycles.reshape(1, n)
    return pl.pallas_call(
        _mask_block,
        grid=(batch // rows_per_block,),
        in_specs=[
            pl.BlockSpec((rows_per_block, n), lambda i: (i, 0)),
            pl.BlockSpec((1, n), lambda i: (0, 0)),
            pl.BlockSpec((rows_per_block, 1), lambda i: (i, 0)),
        ],
        out_specs=pl.BlockSpec((rows_per_block, n), lambda i: (i, 0)),
        out_shape=jax.ShapeDtypeStruct((batch, n), x.dtype),
    )(x, duty2, tkeys)


# speedup vs baseline: 2.9606x; 1.2192x over previous
import functools

import jax
import jax.numpy as jnp
from jax import lax
from jax.experimental import pallas as pl
from jax.experimental.pallas import tpu as pltpu
from jax.experimental.pallas import tpu_sc as plsc

_L = 16


def _skey(f):
    i = lax.bitcast_convert_type(f, jnp.int32)
    return i ^ (lax.shift_right_arithmetic(i, 31) & jnp.int32(0x7FFFFFFF))


def _sc_select_kernel(batch, n, k, rows_per_worker, interpret=False):
    nvec = n // _L
    nbins = 1024
    nhist_vec = nbins // _L
    shift = 22
    mesh = plsc.VectorSubcoreMesh(core_axis_name="c", subcore_axis_name="s")
    num_workers = batch // rows_per_worker

    @functools.partial(
        pl.kernel,
        out_type=jax.ShapeDtypeStruct((num_workers, _L), jnp.int32),
        mesh=mesh,
        scratch_types=[
            pltpu.VMEM((n,), jnp.float32),
            pltpu.VMEM((n,), jnp.int32),
            pltpu.VMEM((n,), jnp.float32),
            pltpu.VMEM((nbins * _L,), jnp.int32),
            pltpu.VMEM((_L,), jnp.int32),
        ],
        compiler_params=pltpu.CompilerParams(needs_layout_passes=False),
        interpret=interpret,
    )
    def sc_select(x_hbm, duty_hbm, out_hbm, buf, cand, fac, hist, tbuf):
        wid = lax.axis_index("s") * 2 + lax.axis_index("c")
        lane = lax.iota(jnp.int32, _L)
        ones = jnp.ones((_L,), jnp.int32)
        zeros = jnp.zeros((_L,), jnp.int32)

        pltpu.sync_copy(duty_hbm, fac)

        @plsc.parallel_loop(0, nvec, unroll=8)
        def _(i):
            sl = pl.ds(i * _L, _L)
            fac[sl] = jnp.exp(-fac[sl])

        tvec = zeros
        for r in range(rows_per_worker):
            row = wid * rows_per_worker + r
            pltpu.sync_copy(x_hbm.at[row], buf)

            @plsc.parallel_loop(0, nbins, unroll=8)
            def _(j):
                hist[pl.ds(j * _L, _L)] = zeros

            @plsc.parallel_loop(0, nvec, unroll=8)
            def _(i):
                sl = pl.ds(i * _L, _L)
                key = _skey(buf[sl] * fac[sl])
                buf[sl] = plsc.bitcast(key, jnp.float32)
                bin_ = lax.shift_right_arithmetic(key, shift) + jnp.int32(512)
                plsc.addupdate_scatter(hist, [bin_ * _L + lane], ones)

            def bf(t, carry):
                s_above, b_run, a_run = carry
                j = nhist_vec - 1 - t
                base = (jnp.int32(j * _L) + lane) * _L
                tot = zeros
                for l in range(_L):
                    tot = tot + plsc.load_gather(hist, [base + jnp.int32(l)])
                s = lax.rev(jnp.cumsum(lax.rev(tot, (0,))), (0,)) + s_above
                m = s >= jnp.int32(k)
                candb = jnp.max(
                    jnp.where(m, jnp.int32(j * _L) + lane, jnp.int32(-1)))
                canda = jnp.min(jnp.where(m, s - tot, jnp.int32(2**30)))
                a_new = jnp.where(candb > b_run, canda, a_run)
                b_new = jnp.maximum(b_run, candb)
                return s_above + jnp.sum(tot), b_new, a_new

            _, b_star, a_above = lax.fori_loop(
                0, nhist_vec, bf, (jnp.int32(0), jnp.int32(-1), jnp.int32(0))
            )
            rank = jnp.int32(k) - a_above
            bin_lo = lax.shift_left(b_star - jnp.int32(512), jnp.int32(shift))

            @plsc.parallel_loop(0, nvec, unroll=4, carry=zeros)
            def off_v(i, off):
                sl = pl.ds(i * _L, _L)
                key = plsc.bitcast(buf[sl], jnp.int32)
                m = (lax.shift_right_arithmetic(key, shift)
                     + jnp.int32(512)) == b_star
                plsc.store_compressed(cand.at[pl.ds(off[0], _L)], key, mask=m)
                return off + plsc.all_reduce_population_count(m)

            ncand = off_v[0]
            ncvec = (ncand + jnp.int32(_L - 1)) // jnp.int32(_L)

            rank_v = jnp.broadcast_to(rank, (_L,))
            ncand_v = jnp.broadcast_to(ncand, (_L,))

            def bs(b, prefix):
                trial = prefix + lax.shift_left(
                    jnp.int32(1), jnp.int32(shift - 1) - b)
                trial_v = jnp.broadcast_to(trial, (_L,))

                @plsc.parallel_loop(0, ncvec, unroll=4, carry=zeros)
                def cnt(i, c):
                    inb = (i * _L + lane) < ncand_v
                    ge = (cand[pl.ds(i * _L, _L)] >= trial_v) & inb
                    return c + plsc.all_reduce_population_count(ge)

                return jnp.where(jnp.max(cnt) >= rank, trial, prefix)

            tkey = lax.fori_loop(0, shift, bs, bin_lo)
            tvec = jnp.where(lane == r, tkey, tvec)

        tbuf[...] = tvec
        pltpu.sync_copy(tbuf, out_hbm.at[wid])

    return sc_select


def _mask_block(x_ref, duty_ref, t_ref, o_ref):
    x = x_ref[...]
    boosted = x * jnp.exp(-duty_ref[...])
    skey = _skey(boosted)
    o_ref[...] = jnp.where(skey < t_ref[...], jnp.zeros_like(x), x)


def kernel(x, duty_cycles):
    batch, n = x.shape
    k = int(round(n * 0.25))
    rows_per_worker = batch // 32

    sc_out = _sc_select_kernel(batch, n, k, rows_per_worker)(x, duty_cycles)
    tkeys = sc_out[:, :rows_per_worker].reshape(batch, 1)

    rows_per_block = 8
    duty2 = duty_cycles.reshape(1, n)
    return pl.pallas_call(
        _mask_block,
        grid=(batch // rows_per_block,),
        in_specs=[
            pl.BlockSpec((rows_per_block, n), lambda i: (i, 0)),
            pl.BlockSpec((1, n), lambda i: (0, 0)),
            pl.BlockSpec((rows_per_block, 1), lambda i: (i, 0)),
        ],
        out_specs=pl.BlockSpec((rows_per_block, n), lambda i: (i, 0)),
        out_shape=jax.ShapeDtypeStruct((batch, n), x.dtype),
    )(x, duty2, tkeys)
